# baseline (speedup 1.0000x reference)
import functools

import jax
import jax.numpy as jnp
from jax.experimental import pallas as pl
from jax.experimental.pallas import tpu as pltpu


def _mlp_head_kernel(x_ref, w1_hbm, w234_hbm, b_hbm, wh_hbm, bh_hbm,
                     bc_ref, fc_ref,
                     w1_s, w234_s, b_s, wh_s, bh_s, sems, *, Lb):
    @pl.when(pl.program_id(0) == 0)
    def _load_weights():
        copies = [
            pltpu.make_async_copy(w1_hbm, w1_s, sems.at[0]),
            pltpu.make_async_copy(w234_hbm, w234_s, sems.at[1]),
            pltpu.make_async_copy(b_hbm, b_s, sems.at[2]),
            pltpu.make_async_copy(wh_hbm, wh_s, sems.at[3]),
            pltpu.make_async_copy(bh_hbm, bh_s, sems.at[4]),
        ]
        for c in copies:
            c.start()
        for c in copies:
            c.wait()

    nt = (((1,), (1,)), ((), ()))
    xb = x_ref[...].astype(jnp.bfloat16)
    h = jax.lax.dot_general(xb, w1_s[...].astype(jnp.bfloat16), nt,
                            preferred_element_type=jnp.float32)
    h = jnp.maximum(h + b_s[0], 0.0).astype(jnp.bfloat16)
    for i in range(3):
        h = jax.lax.dot_general(h, w234_s[i].astype(jnp.bfloat16), nt,
                                preferred_element_type=jnp.float32)
        h = jnp.maximum(h + b_s[i + 1], 0.0).astype(jnp.bfloat16)
    out = jax.lax.dot_general(h, wh_s[...].astype(jnp.bfloat16), nt,
                              preferred_element_type=jnp.float32)
    out = out + bh_s[...]
    bc_ref[...] = out[:, :Lb]
    fc_ref[...] = out[:, Lb:]


def kernel(x, w1_t, w234_t, b1234, wh_t, bh):
    B, Lb = x.shape
    Lf = wh_t.shape[0] - Lb

    b = jnp.transpose(b1234, (0, 2, 1))
    bh_row = bh.T

    tile_b = 2048 if B % 2048 == 0 else B

    hbm = functools.partial(pl.BlockSpec, memory_space=pltpu.MemorySpace.HBM)
    in_specs = [
        pl.BlockSpec((tile_b, Lb), lambda i: (i, 0)),
        hbm(), hbm(), hbm(), hbm(), hbm(),
    ]
    out_specs = (
        pl.BlockSpec((tile_b, Lb), lambda i: (i, 0)),
        pl.BlockSpec((tile_b, Lf), lambda i: (i, 0)),
    )
    scratch_shapes = [
        pltpu.VMEM(w1_t.shape, jnp.float32),
        pltpu.VMEM(w234_t.shape, jnp.float32),
        pltpu.VMEM(b.shape, jnp.float32),
        pltpu.VMEM(wh_t.shape, jnp.float32),
        pltpu.VMEM(bh_row.shape, jnp.float32),
        pltpu.SemaphoreType.DMA((5,)),
    ]

    return pl.pallas_call(
        functools.partial(_mlp_head_kernel, Lb=Lb),
        out_shape=(jax.ShapeDtypeStruct((B, Lb), jnp.float32),
                   jax.ShapeDtypeStruct((B, Lf), jnp.float32)),
        grid=(B // tile_b,),
        in_specs=in_specs,
        out_specs=out_specs,
        scratch_shapes=scratch_shapes,
        compiler_params=pltpu.CompilerParams(
            dimension_semantics=("parallel",),
            vmem_limit_bytes=60 * 1024 * 1024,
            disable_bounds_checks=True,
            disable_semaphore_checks=True),
    )(x, w1_t, w234_t, b, wh_t, bh_row)
